# baseline (speedup 1.0000x reference)
import functools

import jax
import jax.numpy as jnp
from jax import lax
from jax.experimental import pallas as pl
from jax.experimental.pallas import tpu as pltpu
from jax.experimental.pallas import tpu_sc as plsc

BATCH = 16384
EMBED_DIM = 32
LANES = 16
NROWS = 1000000
NBLK = 128
FG = 8
NFG = EMBED_DIM // FG

_info = plsc.get_sparse_core_info()
NC, NS = _info.num_cores, _info.num_subcores
NW = NC * NS
BPW = BATCH // NW
CH = 4
NCHUNK = BPW // CH
TPC = CH * NFG


def _mf_body(uids, iids, eut, eit, bu, bi, gb, out,
             uid_v, iid_v, utiles, itiles,
             bu_v, bi_v, gb_v, outf_v, out_v,
             sem_u, sem_i, sem_bu, sem_bi):
    wid = lax.axis_index("s") * NC + lax.axis_index("c")
    base = wid * BPW

    pltpu.sync_copy(uids.at[pl.ds(base, BPW)], uid_v)
    pltpu.sync_copy(iids.at[pl.ds(base, BPW)], iid_v)

    cbu = pltpu.async_copy(bu.at[uid_v], bu_v, sem_bu)
    cbi = pltpu.async_copy(bi.at[iid_v], bi_v, sem_bi)
    pltpu.sync_copy(gb, gb_v.at[pl.ds(0, 1)])

    lanes = lax.iota(jnp.int32, LANES)
    quad = jnp.right_shift(lanes, 2)

    def read_ids(c):
        uvec = plsc.load_gather(uid_v, [c * CH + quad])
        ivec = plsc.load_gather(iid_v, [c * CH + quad])
        return uvec, ivec

    def issue_u(c, buf):
        uvec, _ = read_ids(c)
        for n in range(CH):
            uid = uvec[4 * n]
            uoff = pl.multiple_of(jnp.right_shift(uid, 7) * NBLK, NBLK)
            pltpu.async_copy(
                eut.at[pl.ds(0, EMBED_DIM), pl.ds(uoff, NBLK)],
                utiles.at[buf, n], sem_u)

    def issue_i(c, buf):
        _, ivec = read_ids(c)
        for n in range(CH):
            iid = ivec[4 * n]
            ioff = pl.multiple_of(jnp.right_shift(iid, 7) * NBLK, NBLK)
            pltpu.async_copy(
                eit.at[pl.ds(0, EMBED_DIM), pl.ds(ioff, NBLK)],
                itiles.at[buf, n], sem_i)

    for p in range(4):
        issue_u(p, p)
    for p in range(3):
        issue_i(p, p)

    def chunk_body(g, acc):
        buf_u = g & 3
        buf_i = lax.rem(g, 3)
        pltpu.make_async_copy(
            eut.at[pl.ds(0, EMBED_DIM), pl.ds(0, CH * NBLK)],
            utiles.at[buf_u], sem_u).wait()
        pltpu.make_async_copy(
            eit.at[pl.ds(0, EMBED_DIM), pl.ds(0, CH * NBLK)],
            itiles.at[buf_i], sem_i).wait()

        bufv_u = jnp.full((LANES,), buf_u, jnp.int32)
        bufv_i = jnp.full((LANES,), buf_i, jnp.int32)
        uvec, ivec = read_ids(g)
        for n in range(CH):
            nv = jnp.full((LANES,), n, jnp.int32)
            ucol = jnp.full((LANES,), uvec[4 * n] & (NBLK - 1), jnp.int32)
            icol = jnp.full((LANES,), ivec[4 * n] & (NBLK - 1), jnp.int32)
            u_lo = plsc.load_gather(utiles, [bufv_u, nv, lanes, ucol])
            u_hi = plsc.load_gather(utiles, [bufv_u, nv, LANES + lanes, ucol])
            i_lo = plsc.load_gather(itiles, [bufv_i, nv, lanes, icol])
            i_hi = plsc.load_gather(itiles, [bufv_i, nv, LANES + lanes, icol])
            s = jnp.sum(u_lo * i_lo + u_hi * i_hi)
            acc = jnp.where(lanes == (g & 3) * CH + n, s, acc)

        @pl.when((g & 3) == 3)
        def _():
            outf_v[pl.ds(jnp.right_shift(g, 2) * LANES, LANES)] = acc

        @pl.when(g < NCHUNK - 4)
        def _():
            issue_u(g + 4, buf_u)

        @pl.when(g < NCHUNK - 3)
        def _():
            issue_i(g + 3, buf_i)

        return jnp.where((g & 3) == 3, jnp.zeros((LANES,), jnp.float32), acc)

    lax.fori_loop(0, NCHUNK, chunk_body, jnp.zeros((LANES,), jnp.float32))

    cbu.wait()
    cbi.wait()
    gbs = gb_v[...][0]

    def finish(k, carry):
        o = k * LANES
        out_v[pl.ds(o, LANES)] = (outf_v[pl.ds(o, LANES)]
                                  + bu_v[pl.ds(o, LANES)]
                                  + bi_v[pl.ds(o, LANES)] + gbs)
        return carry

    lax.fori_loop(0, BPW // LANES, finish, 0)
    pltpu.sync_copy(out_v, out.at[pl.ds(base, BPW)])


@jax.jit
def _mf(user_ids, item_ids, embedding_users, embedding_items,
        bias_users, bias_items, global_bias):
    mesh = plsc.VectorSubcoreMesh(core_axis_name="c", subcore_axis_name="s")
    run = pl.kernel(
        _mf_body,
        mesh=mesh,
        out_type=jax.ShapeDtypeStruct((BATCH,), jnp.float32),
        compiler_params=pltpu.CompilerParams(
            needs_layout_passes=False, use_tc_tiling_on_sc=True),
        scratch_types=[
            pltpu.VMEM((BPW,), jnp.int32),
            pltpu.VMEM((BPW,), jnp.int32),
            pltpu.VMEM((4, CH, EMBED_DIM, NBLK), jnp.float32),
            pltpu.VMEM((3, CH, EMBED_DIM, NBLK), jnp.float32),
            pltpu.VMEM((BPW,), jnp.float32),
            pltpu.VMEM((BPW,), jnp.float32),
            pltpu.VMEM((LANES,), jnp.float32),
            pltpu.VMEM((BPW,), jnp.float32),
            pltpu.VMEM((BPW,), jnp.float32),
            pltpu.SemaphoreType.DMA,
            pltpu.SemaphoreType.DMA,
            pltpu.SemaphoreType.DMA,
            pltpu.SemaphoreType.DMA,
        ],
    )
    return run(user_ids, item_ids, embedding_users.T, embedding_items.T,
               bias_users, bias_items, global_bias)


def kernel(user_ids, item_ids, embedding_users, embedding_items,
           bias_users, bias_items, global_bias):
    return _mf(user_ids, item_ids, embedding_users, embedding_items,
               bias_users, bias_items, global_bias)
